# baseline (speedup 1.0000x reference)
import functools

import jax
import jax.numpy as jnp
from jax import lax
from jax.experimental import pallas as pl
from jax.experimental.pallas import tpu as pltpu
from jax.experimental.pallas import tpu_sc as plsc

B, C, D, H, W = 4, 64, 32, 32, 32
DHW = D * H * W
N = 50000
TROWS = B * DHW // 2

NC, NS, L = 2, 16, 16
NW = NC * NS
TILES_PER_B = NW // B
CHUNK = 128
NCHUNKS = 50
PTS = NCHUNKS * CHUNK
STRIDE = 6272
LAST_START = 43648
NG = CHUNK // L
HPTS = PTS // 2
DEPTH = 3


def _transpose_chunk(rows, trans, offbuf, base):
    iota = lax.iota(jnp.int32, L)
    ridx = [iota + j * L for j in range(NG)]
    offs = [offbuf[pl.ds(base + j * L, L)] for j in range(NG)]
    diag = [(iota + k) & (L - 1) for k in range(L)]

    def cb_body(cb, carry):
        c0 = cb * L
        for j in range(NG):
            for k in range(L):
                crel = diag[k] + c0
                v = plsc.load_gather(rows, [ridx[j], offs[j] + crel])
                plsc.store_scatter(trans, [crel, ridx[j]], v)
        return carry
    lax.fori_loop(0, C // L, cb_body, 0)


def _sc_body(table_hbm, grid_hbm, out_hbm,
             gbuf, idxbuf, offbuf,
             rows0, rows1, rows2, trans0, trans1, trans2,
             sg0, sg1, sg2, so0, so1, so2):
    rows = (rows0, rows1, rows2)
    sg = (sg0, sg1, sg2)
    trans = (trans0, trans1, trans2)
    so = (so0, so1, so2)

    wid = lax.axis_index("s") * NC + lax.axis_index("c")
    b = wid // TILES_PER_B
    r = wid % TILES_PER_B
    start = jnp.minimum(r * STRIDE, LAST_START)
    row_base = b * DHW

    for coord, scale in ((0, H * W), (1, W), (2, 1)):
        for half in range(2):
            h0 = half * HPTS
            pltpu.sync_copy(grid_hbm.at[coord, :, pl.ds(start + h0, HPTS)],
                            gbuf)

            def cpass(k, carry, coord=coord, scale=scale, h0=h0):
                for g in range(NG):
                    p0 = k * CHUNK + g * L
                    cv = gbuf[b, pl.ds(p0, L)]
                    ci = jnp.clip(cv.astype(jnp.int32), 0, D - 1) * scale
                    q0 = h0 + p0
                    if coord == 0:
                        idxbuf[pl.ds(q0, L)] = ci + row_base
                    elif coord == 1:
                        idxbuf[pl.ds(q0, L)] = idxbuf[pl.ds(q0, L)] + ci
                    else:
                        v = idxbuf[pl.ds(q0, L)] + ci
                        idxbuf[pl.ds(q0, L)] = lax.shift_right_logical(v, 1)
                        offbuf[pl.ds(q0, L)] = lax.shift_left(v & 1, 6)
                return carry
            lax.fori_loop(0, NCHUNKS // 2, cpass, 0)

    def fire_gather(k, rbuf, sem):
        pltpu.async_copy(table_hbm.at[idxbuf.at[pl.ds(k * CHUNK, CHUNK)]],
                         rbuf, sem)

    def wait_gather(rbuf, sem):
        pltpu.make_async_copy(table_hbm.at[idxbuf.at[pl.ds(0, CHUNK)]],
                              rbuf, sem).wait()

    def fire_write(k, tbuf, sem):
        n0 = start + k * CHUNK
        pltpu.async_copy(tbuf, out_hbm.at[b, :, pl.ds(n0, CHUNK)], sem)

    def wait_write(tbuf, sem):
        pltpu.make_async_copy(tbuf, out_hbm.at[b, :, pl.ds(0, CHUNK)],
                              sem).wait()

    for k in range(DEPTH):
        fire_gather(k, rows[k], sg[k])

    def pipe_body(t, carry):
        k0 = DEPTH * t
        for i in range(DEPTH):
            k = k0 + i
            wait_gather(rows[i], sg[i])

            @pl.when(t > 0)
            def _(i=i):
                wait_write(trans[i], so[i])
            _transpose_chunk(rows[i], trans[i], offbuf, k * CHUNK)

            @pl.when(k + DEPTH < NCHUNKS)
            def _(k=k, i=i):
                fire_gather(k + DEPTH, rows[i], sg[i])
            fire_write(k, trans[i], so[i])
        return carry
    lax.fori_loop(0, (NCHUNKS - 2) // DEPTH, pipe_body, 0)

    for k in (NCHUNKS - 2, NCHUNKS - 1):
        i = k % DEPTH
        wait_gather(rows[i], sg[i])
        wait_write(trans[i], so[i])
        _transpose_chunk(rows[i], trans[i], offbuf, k * CHUNK)
        fire_write(k, trans[i], so[i])
    for i in range(DEPTH):
        wait_write(trans[i], so[i])


_sc_gather = functools.partial(
    pl.kernel,
    out_type=jax.ShapeDtypeStruct((B, C, N), jnp.float32),
    mesh=plsc.VectorSubcoreMesh(core_axis_name="c", subcore_axis_name="s"),
    scratch_types=[
        pltpu.VMEM((B, HPTS), jnp.float32),
        pltpu.VMEM((PTS,), jnp.int32),
        pltpu.VMEM((PTS,), jnp.int32),
        pltpu.VMEM((CHUNK, 2 * C), jnp.float32),
        pltpu.VMEM((CHUNK, 2 * C), jnp.float32),
        pltpu.VMEM((CHUNK, 2 * C), jnp.float32),
        pltpu.VMEM((C, CHUNK), jnp.float32),
        pltpu.VMEM((C, CHUNK), jnp.float32),
        pltpu.VMEM((C, CHUNK), jnp.float32),
        pltpu.SemaphoreType.DMA,
        pltpu.SemaphoreType.DMA,
        pltpu.SemaphoreType.DMA,
        pltpu.SemaphoreType.DMA,
        pltpu.SemaphoreType.DMA,
        pltpu.SemaphoreType.DMA,
    ],
    compiler_params=pltpu.CompilerParams(use_tc_tiling_on_sc=True,
                                         needs_layout_passes=False,
                                         disable_bounds_checks=True),
)(_sc_body)


def kernel(input_feats, sampling_grid):
    assert input_feats.shape == (B, C, D, H, W), input_feats.shape
    assert sampling_grid.shape == (B, N, 3), sampling_grid.shape
    table = input_feats.transpose(0, 2, 3, 4, 1).reshape(TROWS, 2 * C)
    planes = sampling_grid.transpose(2, 0, 1)
    return _sc_gather(table, planes)
